# Initial kernel scaffold; baseline (speedup 1.0000x reference)
import jax
import jax.numpy as jnp
from jax.experimental import pallas as pl


def kernel(indices, cond, table):
    raise NotImplementedError("write your pallas kernel here")


# speedup vs baseline: 2.1812x; 2.1812x over previous
import jax
import jax.numpy as jnp
from jax import lax
from jax.experimental import pallas as pl
from jax.experimental.pallas import tpu as pltpu
from jax.experimental.pallas import tpu_sc as plsc

_B = 16384
_D = 128
_NC = 2
_NS = 16
_NW = _NC * _NS
_BPW = _B // _NW


def _sc_body(idx_hbm, cond_hbm, table_hbm, out_hbm, idx_v, table_v, buf_v):
    wid = lax.axis_index("s") * _NC + lax.axis_index("c")
    base = wid * _BPW
    pltpu.sync_copy(table_hbm, table_v)
    pltpu.sync_copy(idx_hbm.at[pl.ds(base, _BPW)], idx_v)
    pltpu.sync_copy(cond_hbm.at[pl.ds(base * _D, _BPW * _D)], buf_v)

    t1 = [table_v[pl.ds(_D + 16 * j, 16)] for j in range(8)]
    d = [table_v[pl.ds(16 * j, 16)] - t1[j] for j in range(8)]

    def group_body(g, carry):
        gb = g * 16
        fv = 1.0 - idx_v[pl.ds(gb, 16)].astype(jnp.float32)
        for r in range(16):
            f = lax.broadcast_in_dim(lax.slice(fv, (r,), (r + 1,)), (16,), (0,))
            rb = (gb + r) * _D
            for j in range(8):
                off = pl.ds(rb + 16 * j, 16)
                buf_v[off] = buf_v[off] + (t1[j] + f * d[j])
        return carry

    lax.fori_loop(0, _BPW // 16, group_body, 0)
    pltpu.sync_copy(buf_v, out_hbm.at[pl.ds(base * _D, _BPW * _D)])


@jax.jit
def _run(idx, cond_flat, table_flat):
    mesh = plsc.VectorSubcoreMesh(core_axis_name="c", subcore_axis_name="s")
    return pl.kernel(
        _sc_body,
        out_type=jax.ShapeDtypeStruct((_B * _D,), jnp.float32),
        mesh=mesh,
        scratch_types=[
            pltpu.VMEM((_BPW,), jnp.int32),
            pltpu.VMEM((2 * _D,), jnp.float32),
            pltpu.VMEM((_BPW * _D,), jnp.float32),
        ],
    )(idx, cond_flat, table_flat)


def kernel(indices, cond, table):
    idx = indices.astype(jnp.int32)
    out_flat = _run(idx, cond.reshape(-1), table.reshape(-1))
    return out_flat.reshape(_B, _D)
